# baseline (speedup 1.0000x reference)
import functools

import jax
import jax.numpy as jnp
from jax import lax
from jax.experimental import pallas as pl
from jax.experimental.pallas import tpu as pltpu
from jax.experimental.pallas import tpu_sc as plsc

D_MODEL = 64
SCALE = 0.125


@jax.jit
def _embed(x, table):
    idx = x.reshape(-1)
    B = idx.shape[0]

    info = plsc.get_sparse_core_info()
    NC, NS = info.num_cores, info.num_subcores
    NW = NC * NS
    b_per_w = B // NW
    C = 512
    n_chunks = b_per_w // C

    mesh = plsc.VectorSubcoreMesh(core_axis_name="c", subcore_axis_name="s")

    @functools.partial(
        pl.kernel,
        mesh=mesh,
        compiler_params=pltpu.CompilerParams(use_tc_tiling_on_sc=False),
        out_type=jax.ShapeDtypeStruct((B, D_MODEL), jnp.float32),
        scratch_types=[
            pltpu.VMEM((C,), jnp.int32),
            pltpu.VMEM((C, D_MODEL), jnp.float32),
            pltpu.SemaphoreType.DMA,
        ],
    )
    def sc_kernel(table_hbm, idx_hbm, out_hbm, idx_v, rows_v, sem):
        wid = lax.axis_index("s") * NC + lax.axis_index("c")
        base = wid * b_per_w

        def chunk_body(g, carry):
            off = base + g * C
            pltpu.sync_copy(idx_hbm.at[pl.ds(off, C)], idx_v)
            pltpu.async_copy(table_hbm.at[idx_v], rows_v, sem).wait()

            def scale_body(i, c):
                for j in range(D_MODEL // 16):
                    sl = pl.ds(j * 16, 16)
                    rows_v[i, sl] = rows_v[i, sl] * SCALE
                return c

            lax.fori_loop(0, C, scale_body, 0)
            pltpu.sync_copy(rows_v, out_hbm.at[pl.ds(off, C)])
            return carry

        lax.fori_loop(0, n_chunks, chunk_body, 0)

    out = sc_kernel(table, idx)
    return out.reshape(x.shape + (D_MODEL,))


def kernel(x, table):
    return _embed(x, table)


# speedup vs baseline: 1.1386x; 1.1386x over previous
import functools

import jax
import jax.numpy as jnp
from jax import lax
from jax.experimental import pallas as pl
from jax.experimental.pallas import tpu as pltpu
from jax.experimental.pallas import tpu_sc as plsc

D_MODEL = 64
SCALE = 0.125
NBUF = 4
C = 256
U = 4


@jax.jit
def _embed(x, table):
    idx = x.reshape(-1)
    B = idx.shape[0]

    info = plsc.get_sparse_core_info()
    NC, NS = info.num_cores, info.num_subcores
    NW = NC * NS
    b_per_w = B // NW
    assert b_per_w * NW == B
    n_chunks = b_per_w // C
    assert n_chunks * C == b_per_w
    assert n_chunks >= 6 and (n_chunks - 4) % NBUF == 0
    n_outer = (n_chunks - 4) // NBUF

    mesh = plsc.VectorSubcoreMesh(core_axis_name="c", subcore_axis_name="s")

    @functools.partial(
        pl.kernel,
        mesh=mesh,
        compiler_params=pltpu.CompilerParams(use_tc_tiling_on_sc=False),
        out_type=jax.ShapeDtypeStruct((B, D_MODEL), jnp.float32),
        scratch_types=(
            [pltpu.VMEM((b_per_w,), jnp.int32)]
            + [pltpu.VMEM((C, D_MODEL), jnp.float32) for _ in range(NBUF)]
            + [pltpu.SemaphoreType.DMA for _ in range(2 * NBUF)]
        ),
    )
    def sc_kernel(table_hbm, idx_hbm, out_hbm, idx_all, *bufs):
        rows = bufs[:NBUF]
        gsem = bufs[NBUF:2 * NBUF]
        wsem = bufs[2 * NBUF:]
        wid = lax.axis_index("s") * NC + lax.axis_index("c")
        base = wid * b_per_w

        def gather_desc(g, b):
            isl = idx_all.at[pl.ds(g * C, C)]
            return pltpu.make_async_copy(table_hbm.at[isl], rows[b], gsem[b])

        def wb_desc(g, b):
            return pltpu.make_async_copy(
                rows[b], out_hbm.at[pl.ds(base + g * C, C)], wsem[b])

        def scale(b):
            rb = rows[b]

            def sbody(i, c):
                for u in range(U):
                    for j in range(D_MODEL // 16):
                        sl = pl.ds(j * 16, 16)
                        rb[i * U + u, sl] = rb[i * U + u, sl] * SCALE
                return c

            lax.fori_loop(0, C // U, sbody, 0)

        pltpu.sync_copy(idx_hbm.at[pl.ds(base, b_per_w)], idx_all)

        gather_desc(0, 0).start()
        gather_desc(1, 1).start()

        for g in (0, 1):
            b = g % NBUF
            gather_desc(g, b).wait()
            scale(b)
            wb_desc(g, b).start()
            gather_desc(g + 2, (g + 2) % NBUF).start()

        def outer(go, c):
            for k in range(NBUF):
                g = 2 + go * NBUF + k
                b = (2 + k) % NBUF
                b2 = k
                gather_desc(g, b).wait()
                scale(b)
                wb_desc(g, b).start()
                wb_desc(g - 2, b2).wait()
                gather_desc(g + 2, b2).start()
            return c

        lax.fori_loop(0, n_outer, outer, 0)

        for g in (n_chunks - 2, n_chunks - 1):
            b = g % NBUF
            gather_desc(g, b).wait()
            scale(b)
            wb_desc(g, b).start()

        for g in range(n_chunks - NBUF, n_chunks):
            wb_desc(g, g % NBUF).wait()

    out = sc_kernel(table, idx)
    return out.reshape(x.shape + (D_MODEL,))


def kernel(x, table):
    return _embed(x, table)


# speedup vs baseline: 1.5137x; 1.3295x over previous
import functools

import jax
import jax.numpy as jnp
from jax import lax
from jax.experimental import pallas as pl
from jax.experimental.pallas import tpu as pltpu
from jax.experimental.pallas import tpu_sc as plsc

D_MODEL = 64
SCALE = 0.125
NBUF = 4
C = 256
U = 4


@jax.jit
def _embed(x, table):
    idx = x.reshape(-1)
    B = idx.shape[0]

    info = plsc.get_sparse_core_info()
    NC, NS = info.num_cores, info.num_subcores
    NW = NC * NS
    b_per_w = B // NW
    assert b_per_w * NW == B
    n_chunks = b_per_w // C
    assert n_chunks * C == b_per_w
    assert n_chunks >= 6 and (n_chunks - 4) % NBUF == 0
    n_outer = (n_chunks - 4) // NBUF

    mesh = plsc.VectorSubcoreMesh(core_axis_name="c", subcore_axis_name="s")

    @functools.partial(
        pl.kernel,
        mesh=mesh,
        compiler_params=pltpu.CompilerParams(use_tc_tiling_on_sc=False),
        out_type=jax.ShapeDtypeStruct((B, 128), jnp.float32),
        scratch_types=(
            [pltpu.VMEM((b_per_w,), jnp.int32)]
            + [pltpu.VMEM((C, D_MODEL), jnp.float32) for _ in range(NBUF)]
            + [pltpu.SemaphoreType.DMA for _ in range(2 * NBUF)]
        ),
    )
    def sc_kernel(table_hbm, idx_hbm, out_hbm, idx_all, *bufs):
        rows = bufs[:NBUF]
        gsem = bufs[NBUF:2 * NBUF]
        wsem = bufs[2 * NBUF:]
        wid = lax.axis_index("s") * NC + lax.axis_index("c")
        base = wid * b_per_w

        def gather_desc(g, b):
            isl = idx_all.at[pl.ds(g * C, C)]
            return pltpu.make_async_copy(table_hbm.at[isl], rows[b], gsem[b])

        def wb_desc(g, b):
            return pltpu.make_async_copy(
                rows[b],
                out_hbm.at[pl.ds(base + g * C, C), pl.ds(0, D_MODEL)],
                wsem[b])

        def scale(b):
            rb = rows[b]

            def sbody(i, c):
                for u in range(U):
                    for j in range(D_MODEL // 16):
                        sl = pl.ds(j * 16, 16)
                        rb[i * U + u, sl] = rb[i * U + u, sl] * SCALE
                return c

            lax.fori_loop(0, C // U, sbody, 0)

        pltpu.sync_copy(idx_hbm.at[pl.ds(base, b_per_w)], idx_all)

        gather_desc(0, 0).start()
        gather_desc(1, 1).start()

        for g in (0, 1):
            b = g % NBUF
            gather_desc(g, b).wait()
            scale(b)
            wb_desc(g, b).start()
            gather_desc(g + 2, (g + 2) % NBUF).start()

        def outer(go, c):
            for k in range(NBUF):
                g = 2 + go * NBUF + k
                b = (2 + k) % NBUF
                b2 = k
                gather_desc(g, b).wait()
                scale(b)
                wb_desc(g, b).start()
                wb_desc(g - 2, b2).wait()
                gather_desc(g + 2, b2).start()
            return c

        lax.fori_loop(0, n_outer, outer, 0)

        for g in (n_chunks - 2, n_chunks - 1):
            b = g % NBUF
            gather_desc(g, b).wait()
            scale(b)
            wb_desc(g, b).start()

        for g in range(n_chunks - NBUF, n_chunks):
            wb_desc(g, g % NBUF).wait()

    out = sc_kernel(table, idx)
    return out[:, :D_MODEL].reshape(x.shape + (D_MODEL,))


def kernel(x, table):
    return _embed(x, table)
